# Initial kernel scaffold; baseline (speedup 1.0000x reference)
import jax
import jax.numpy as jnp
from jax.experimental import pallas as pl


def kernel(user_sequence, recommendations, distance_matrix):
    raise NotImplementedError("write your pallas kernel here")


# speedup vs baseline: 32.2117x; 32.2117x over previous
import functools

import jax
import jax.numpy as jnp
from jax.experimental import pallas as pl
from jax.experimental.pallas import tpu as pltpu

_VP = 1024


def _ild_kernel(rec_ref, d_ref, out_ref):
    rec = rec_ref[...]
    bblk, k = rec.shape
    iota = jax.lax.broadcasted_iota(jnp.int32, (bblk, _VP), 1)

    c = jnp.zeros((bblk, _VP), jnp.float32)
    for i in range(k):
        c = c + (iota == rec[:, i : i + 1]).astype(jnp.float32)
    t = jnp.dot(c, d_ref[...], preferred_element_type=jnp.float32)
    out_ref[...] = jnp.sum(t * c, axis=1)


@jax.jit
def kernel(user_sequence, recommendations, distance_matrix):
    del user_sequence
    b, k = recommendations.shape
    v = distance_matrix.shape[0]
    d_pad = jnp.zeros((_VP, _VP), jnp.float32).at[:v, :v].set(distance_matrix)
    rec = recommendations.astype(jnp.int32)

    bblk = 256
    grid = (b // bblk,)
    distance_sum = pl.pallas_call(
        _ild_kernel,
        grid=grid,
        in_specs=[
            pl.BlockSpec((bblk, k), lambda i: (i, 0)),
            pl.BlockSpec((_VP, _VP), lambda i: (0, 0)),
        ],
        out_specs=pl.BlockSpec((bblk,), lambda i: (i,)),
        out_shape=jax.ShapeDtypeStruct((b,), jnp.float32),
    )(rec, d_pad)
    return distance_sum / (k * (k - 1))


# speedup vs baseline: 50.7554x; 1.5757x over previous
import functools

import jax
import jax.numpy as jnp
from jax.experimental import pallas as pl
from jax.experimental.pallas import tpu as pltpu

_VP = 1024


def _ild_kernel(rec_ref, d_ref, out_ref):
    rec = rec_ref[...].astype(jnp.int16)
    bblk, k = rec.shape
    iota = jax.lax.broadcasted_iota(jnp.int16, (bblk, _VP), 1)

    c16 = jnp.zeros((bblk, _VP), jnp.bfloat16)
    one = jnp.ones((), jnp.bfloat16)
    zero = jnp.zeros((), jnp.bfloat16)
    for i in range(k):
        c16 = c16 + jnp.where(iota == rec[:, i : i + 1], one, zero)
    c = c16.astype(jnp.float32)
    t = jnp.dot(c, d_ref[...], preferred_element_type=jnp.float32)
    out_ref[...] = jnp.sum(t * c, axis=1)


@jax.jit
def kernel(user_sequence, recommendations, distance_matrix):
    del user_sequence
    b, k = recommendations.shape
    v = distance_matrix.shape[0]
    d_pad = jnp.zeros((_VP, _VP), jnp.float32).at[:v, :v].set(distance_matrix)
    rec = recommendations.astype(jnp.int32)

    bblk = 256
    grid = (b // bblk,)
    distance_sum = pl.pallas_call(
        _ild_kernel,
        grid=grid,
        in_specs=[
            pl.BlockSpec((bblk, k), lambda i: (i, 0)),
            pl.BlockSpec((_VP, _VP), lambda i: (0, 0)),
        ],
        out_specs=pl.BlockSpec((bblk,), lambda i: (i,)),
        out_shape=jax.ShapeDtypeStruct((b,), jnp.float32),
    )(rec, d_pad)
    return distance_sum / (k * (k - 1))
